# baseline (speedup 1.0000x reference)
import functools

import jax
import jax.numpy as jnp
from jax import lax
from jax.experimental import pallas as pl
from jax.experimental.pallas import tpu as pltpu
from jax.experimental.pallas import tpu_sc as plsc


def _sc_info():
    try:
        info = plsc.get_sparse_core_info()
        return info.num_cores, info.num_subcores
    except Exception:
        return 2, 16


@functools.lru_cache(maxsize=None)
def _make_sc_gather(V, D, B, R):
    NC, NS = _sc_info()
    NW = NC * NS
    assert B % NW == 0
    b_per_w = B // NW
    assert b_per_w % R == 0
    n_chunks = b_per_w // R

    mesh = plsc.VectorSubcoreMesh(core_axis_name="c", subcore_axis_name="s")

    @functools.partial(
        pl.kernel,
        mesh=mesh,
        out_type=jax.ShapeDtypeStruct((B, D), jnp.float32),
        scratch_types=[
            pltpu.VMEM((R,), jnp.int32),
            pltpu.VMEM((R, D), jnp.float32),
            pltpu.SemaphoreType.DMA,
        ],
        compiler_params=pltpu.CompilerParams(use_tc_tiling_on_sc=False),
    )
    def gather_kernel(table_hbm, idx_hbm, out_hbm, idx_v, rows_v, sem):
        wid = lax.axis_index("s") * NC + lax.axis_index("c")
        base = wid * b_per_w

        def body(ci, carry):
            off = base + ci * R
            pltpu.sync_copy(idx_hbm.at[pl.ds(off, R)], idx_v)
            pltpu.async_copy(table_hbm.at[idx_v], rows_v, sem).wait()
            pltpu.sync_copy(rows_v, out_hbm.at[pl.ds(off, R)])
            return carry

        lax.fori_loop(0, n_chunks, body, 0)

    return gather_kernel


def _coarse_transform(x, W, b):
    N, K = x.shape
    C = W.shape[1]

    def body(x_ref, w_ref, b_ref, o_ref):
        o_ref[...] = (
            jnp.dot(x_ref[...], w_ref[...], preferred_element_type=jnp.float32)
            + b_ref[...]
        )

    return pl.pallas_call(
        body,
        out_shape=jax.ShapeDtypeStruct((N, C), jnp.float32),
    )(x, W, b.reshape(1, C))


def _celu(x):
    return jnp.where(x > 0.0, x, jnp.exp(jnp.minimum(x, 0.0)) - 1.0)


def _fine_layer(g, pos, W1b, W2, b2, Wn, bn, BLK):
    N, cm = g.shape
    c2 = W2.shape[1]
    co = Wn.shape[1]

    def body(g_ref, pos_ref, w1b_ref, w2_ref, b2_ref, wn_ref, bn_ref, o_ref):
        h = jnp.maximum(
            g_ref[...]
            + jnp.dot(pos_ref[...], w1b_ref[...],
                      preferred_element_type=jnp.float32),
            0.0,
        )
        u = jnp.dot(h, w2_ref[...], preferred_element_type=jnp.float32) + b2_ref[...]
        o = _celu(u)
        o_ref[...] = (
            jnp.dot(o, wn_ref[...], preferred_element_type=jnp.float32)
            + bn_ref[...]
        )

    full = lambda a: pl.BlockSpec(a.shape, lambda i: (0, 0))
    return pl.pallas_call(
        body,
        grid=(N // BLK,),
        in_specs=[
            pl.BlockSpec((BLK, cm), lambda i: (i, 0)),
            pl.BlockSpec((BLK, 3), lambda i: (i, 0)),
            full(W1b),
            full(W2),
            pl.BlockSpec((1, c2), lambda i: (0, 0)),
            full(Wn),
            pl.BlockSpec((1, co), lambda i: (0, 0)),
        ],
        out_specs=pl.BlockSpec((BLK, co), lambda i: (i, 0)),
        out_shape=jax.ShapeDtypeStruct((N, co), jnp.float32),
    )(g, pos, W1b, W2, b2.reshape(1, c2), Wn, bn.reshape(1, co))


def kernel(z_what, pos1, pos2, pos3, idx1, idx2, idx3,
           c1_W1, c1_b1, c1_W2, c1_b2,
           c2_W1, c2_b1, c2_W2, c2_b2,
           c3_W1, c3_b1, c3_W2, c3_b2,
           lin_W, lin_b):
    idx1 = idx1.astype(jnp.int32)
    idx2 = idx2.astype(jnp.int32)
    idx3 = idx3.astype(jnp.int32)

    t1 = _coarse_transform(z_what, c1_W1[:128], c1_b1)
    g1 = _make_sc_gather(2048, 128, 65536, 512)(t1, idx3)
    t2 = _fine_layer(g1, pos3, c1_W1[128:], c1_W2, c1_b2,
                     c2_W1[:64], c2_b1, 2048)

    g2 = _make_sc_gather(65536, 32, 262144, 2048)(t2, idx2)
    t3 = _fine_layer(g2, pos2, c2_W1[64:], c2_W2, c2_b2,
                     c3_W1[:32], c3_b1, 2048)

    g3 = _make_sc_gather(262144, 16, 1048576, 4096)(t3, idx1)
    out = _fine_layer(g3, pos1, c3_W1[32:], c3_W2, c3_b2,
                      lin_W, lin_b, 2048)
    return out


# speedup vs baseline: 1.1276x; 1.1276x over previous
import functools

import jax
import jax.numpy as jnp
from jax import lax
from jax.experimental import pallas as pl
from jax.experimental.pallas import tpu as pltpu
from jax.experimental.pallas import tpu_sc as plsc


def _sc_info():
    try:
        info = plsc.get_sparse_core_info()
        return info.num_cores, info.num_subcores
    except Exception:
        return 2, 16


@functools.lru_cache(maxsize=None)
def _make_sc_gather(V, D, B, R):
    NC, NS = _sc_info()
    NW = NC * NS
    assert B % NW == 0
    b_per_w = B // NW
    assert b_per_w % R == 0
    n_chunks = b_per_w // R

    mesh = plsc.VectorSubcoreMesh(core_axis_name="c", subcore_axis_name="s")

    @functools.partial(
        pl.kernel,
        mesh=mesh,
        out_type=jax.ShapeDtypeStruct((B, D), jnp.float32),
        scratch_types=[
            pltpu.VMEM((R,), jnp.int32),
            pltpu.VMEM((R, D), jnp.float32),
            pltpu.SemaphoreType.DMA,
        ],
        compiler_params=pltpu.CompilerParams(use_tc_tiling_on_sc=False),
    )
    def gather_kernel(table_hbm, idx_hbm, out_hbm, idx_v, rows_v, sem):
        wid = lax.axis_index("s") * NC + lax.axis_index("c")
        base = wid * b_per_w

        def body(ci, carry):
            off = base + ci * R
            pltpu.sync_copy(idx_hbm.at[pl.ds(off, R)], idx_v)
            pltpu.async_copy(table_hbm.at[idx_v], rows_v, sem).wait()
            pltpu.sync_copy(rows_v, out_hbm.at[pl.ds(off, R)])
            return carry

        lax.fori_loop(0, n_chunks, body, 0)

    return gather_kernel


def _coarse_transform(x, W, b):
    N, K = x.shape
    C = W.shape[1]

    def body(x_ref, w_ref, b_ref, o_ref):
        o_ref[...] = (
            jnp.dot(x_ref[...], w_ref[...], preferred_element_type=jnp.float32)
            + b_ref[...]
        )

    return pl.pallas_call(
        body,
        out_shape=jax.ShapeDtypeStruct((N, C), jnp.float32),
    )(x, W, b.reshape(1, C))


def _celu(x):
    return jnp.where(x > 0.0, x, jnp.exp(jnp.minimum(x, 0.0)) - 1.0)


def _fine_layer(g, pos, W1b, W2, b2, Wn, bn, BLK):
    N, cm = g.shape
    c2 = W2.shape[1]
    co = Wn.shape[1]

    def body(g_ref, pos_ref, w1b_ref, w2_ref, b2_ref, wn_ref, bn_ref, o_ref):
        h = jnp.maximum(
            g_ref[...]
            + jnp.dot(pos_ref[...], w1b_ref[...],
                      preferred_element_type=jnp.float32),
            0.0,
        )
        u = jnp.dot(h, w2_ref[...], preferred_element_type=jnp.float32) + b2_ref[...]
        o = _celu(u)
        o_ref[...] = (
            jnp.dot(o, wn_ref[...], preferred_element_type=jnp.float32)
            + bn_ref[...]
        )

    full = lambda a: pl.BlockSpec(a.shape, lambda i: (0, 0))
    return pl.pallas_call(
        body,
        grid=(N // BLK,),
        in_specs=[
            pl.BlockSpec((BLK, cm), lambda i: (i, 0)),
            pl.BlockSpec((BLK, 3), lambda i: (i, 0)),
            full(W1b),
            full(W2),
            pl.BlockSpec((1, c2), lambda i: (0, 0)),
            full(Wn),
            pl.BlockSpec((1, co), lambda i: (0, 0)),
        ],
        out_specs=pl.BlockSpec((BLK, co), lambda i: (i, 0)),
        out_shape=jax.ShapeDtypeStruct((N, co), jnp.float32),
    )(g, pos, W1b, W2, b2.reshape(1, c2), Wn, bn.reshape(1, co))


def kernel(z_what, pos1, pos2, pos3, idx1, idx2, idx3,
           c1_W1, c1_b1, c1_W2, c1_b2,
           c2_W1, c2_b1, c2_W2, c2_b2,
           c3_W1, c3_b1, c3_W2, c3_b2,
           lin_W, lin_b):
    idx1 = idx1.astype(jnp.int32)
    idx2 = idx2.astype(jnp.int32)
    idx3 = idx3.astype(jnp.int32)

    t1 = _coarse_transform(z_what, c1_W1[:128], c1_b1)
    g1 = _make_sc_gather(2048, 128, 65536, 512)(t1, idx3)
    t2 = _fine_layer(g1, pos3, c1_W1[128:], c1_W2, c1_b2,
                     c2_W1[:64], c2_b1, 2048)

    g2 = _make_sc_gather(65536, 32, 262144, 2048)(t2, idx2)
    t3 = _fine_layer(g2, pos2, c2_W1[64:], c2_W2, c2_b2,
                     c3_W1[:32], c3_b1, 4096)

    g3 = _make_sc_gather(262144, 16, 1048576, 4096)(t3, idx1)
    out = _fine_layer(g3, pos1, c3_W1[32:], c3_W2, c3_b2,
                      lin_W, lin_b, 8192)
    return out


# speedup vs baseline: 1.1503x; 1.0201x over previous
import functools

import jax
import jax.numpy as jnp
from jax import lax
from jax.experimental import pallas as pl
from jax.experimental.pallas import tpu as pltpu
from jax.experimental.pallas import tpu_sc as plsc


def _sc_info():
    try:
        info = plsc.get_sparse_core_info()
        return info.num_cores, info.num_subcores
    except Exception:
        return 2, 16


@functools.lru_cache(maxsize=None)
def _make_sc_gather(V, D, B, R):
    NC, NS = _sc_info()
    NW = NC * NS
    assert B % NW == 0
    b_per_w = B // NW
    assert b_per_w % R == 0
    n_chunks = b_per_w // R

    mesh = plsc.VectorSubcoreMesh(core_axis_name="c", subcore_axis_name="s")

    @functools.partial(
        pl.kernel,
        mesh=mesh,
        out_type=jax.ShapeDtypeStruct((B, D), jnp.float32),
        scratch_types=[
            pltpu.VMEM((R,), jnp.int32),
            pltpu.VMEM((R, D), jnp.float32),
            pltpu.SemaphoreType.DMA,
        ],
        compiler_params=pltpu.CompilerParams(use_tc_tiling_on_sc=False),
    )
    def gather_kernel(table_hbm, idx_hbm, out_hbm, idx_v, rows_v, sem):
        wid = lax.axis_index("s") * NC + lax.axis_index("c")
        base = wid * b_per_w

        def body(ci, carry):
            off = base + ci * R
            pltpu.sync_copy(idx_hbm.at[pl.ds(off, R)], idx_v)
            pltpu.async_copy(table_hbm.at[idx_v], rows_v, sem).wait()
            pltpu.sync_copy(rows_v, out_hbm.at[pl.ds(off, R)])
            return carry

        lax.fori_loop(0, n_chunks, body, 0)

    return gather_kernel


def _coarse_transform(x, W, b):
    N, K = x.shape
    C = W.shape[1]

    def body(x_ref, w_ref, b_ref, o_ref):
        o_ref[...] = (
            jnp.dot(x_ref[...], w_ref[...], preferred_element_type=jnp.float32)
            + b_ref[...]
        )

    return pl.pallas_call(
        body,
        out_shape=jax.ShapeDtypeStruct((N, C), jnp.float32),
    )(x, W, b.reshape(1, C))


def _celu(x):
    return jnp.where(x > 0.0, x, jnp.exp(jnp.minimum(x, 0.0)) - 1.0)


def _fine_layer(g, pos, W1b, W2, b2, Wn, bn, BLK):
    N, cm = g.shape
    c2 = W2.shape[1]
    co = Wn.shape[1]

    def body(g_ref, pos_ref, w1b_ref, w2_ref, b2_ref, wn_ref, bn_ref, o_ref):
        h = jnp.maximum(
            g_ref[...]
            + jnp.dot(pos_ref[...], w1b_ref[...],
                      preferred_element_type=jnp.float32),
            0.0,
        )
        u = jnp.dot(h, w2_ref[...], preferred_element_type=jnp.float32) + b2_ref[...]
        o = _celu(u)
        o_ref[...] = (
            jnp.dot(o, wn_ref[...], preferred_element_type=jnp.float32)
            + bn_ref[...]
        )

    full = lambda a: pl.BlockSpec(a.shape, lambda i: (0, 0))
    return pl.pallas_call(
        body,
        grid=(N // BLK,),
        in_specs=[
            pl.BlockSpec((BLK, cm), lambda i: (i, 0)),
            pl.BlockSpec((BLK, 3), lambda i: (i, 0)),
            full(W1b),
            full(W2),
            pl.BlockSpec((1, c2), lambda i: (0, 0)),
            full(Wn),
            pl.BlockSpec((1, co), lambda i: (0, 0)),
        ],
        out_specs=pl.BlockSpec((BLK, co), lambda i: (i, 0)),
        out_shape=jax.ShapeDtypeStruct((N, co), jnp.float32),
    )(g, pos, W1b, W2, b2.reshape(1, c2), Wn, bn.reshape(1, co))


def kernel(z_what, pos1, pos2, pos3, idx1, idx2, idx3,
           c1_W1, c1_b1, c1_W2, c1_b2,
           c2_W1, c2_b1, c2_W2, c2_b2,
           c3_W1, c3_b1, c3_W2, c3_b2,
           lin_W, lin_b):
    idx1 = idx1.astype(jnp.int32)
    idx2 = idx2.astype(jnp.int32)
    idx3 = idx3.astype(jnp.int32)

    t1 = _coarse_transform(z_what, c1_W1[:128], c1_b1)
    g1 = _make_sc_gather(2048, 128, 65536, 512)(t1, idx3)
    t2 = _fine_layer(g1, pos3, c1_W1[128:], c1_W2, c1_b2,
                     c2_W1[:64], c2_b1, 2048)

    g2 = _make_sc_gather(65536, 32, 262144, 2048)(t2, idx2)
    t3 = _fine_layer(g2, pos2, c2_W1[64:], c2_W2, c2_b2,
                     c3_W1[:32], c3_b1, 8192)

    g3 = _make_sc_gather(262144, 16, 1048576, 4096)(t3, idx1)
    out = _fine_layer(g3, pos1, c3_W1[32:], c3_W2, c3_b2,
                      lin_W, lin_b, 16384)
    return out
